# Initial kernel scaffold; baseline (speedup 1.0000x reference)
import jax
import jax.numpy as jnp
from jax.experimental import pallas as pl


def kernel(input_ids, embedding_table):
    raise NotImplementedError("write your pallas kernel here")


# speedup vs baseline: 4.6381x; 4.6381x over previous
import functools

import jax
import jax.numpy as jnp
import numpy as np
from jax import lax
from jax.experimental import pallas as pl
from jax.experimental.pallas import tpu as pltpu
from jax.experimental.pallas import tpu_sc as plsc

MODEL_DIM = 2048
MAX_SEQ_LEN = 8192


def _pe_table_np(seq_len: int, model_dim: int) -> np.ndarray:
    positions = np.arange(0, seq_len, dtype=np.float32)[:, None]
    i = np.arange(0, model_dim // 2, dtype=np.float32)
    frequencies = np.power(np.float32(10000.0), 2.0 * i / np.float32(model_dim))
    pe = np.zeros((seq_len, model_dim), dtype=np.float32)
    pe[:, 0::2] = np.sin(positions / frequencies)
    pe[:, 1::2] = np.cos(positions / frequencies)
    return pe


_PE = _pe_table_np(MAX_SEQ_LEN, MODEL_DIM)

_NUM_WORKERS = 32
_CHUNK = 16


def _make_sc_kernel(n_rows: int, d: int, seq_len: int, rows_per_w: int):
    mesh = plsc.VectorSubcoreMesh(core_axis_name="c", subcore_axis_name="s")
    n_chunks = rows_per_w // _CHUNK

    @functools.partial(
        pl.kernel,
        mesh=mesh,
        out_type=jax.ShapeDtypeStruct((n_rows, d), jnp.float32),
        scratch_types=[
            pltpu.VMEM((rows_per_w,), jnp.int32),
            pltpu.VMEM((_CHUNK, d), jnp.float32),
            pltpu.VMEM((_CHUNK, d), jnp.float32),
            pltpu.SemaphoreType.DMA,
            pltpu.SemaphoreType.DMA,
        ],
    )
    def body(ids_hbm, table_hbm, pe_hbm, out_hbm, idx_v, rows_v, pe_v, gsem, psem):
        nc = 2
        wid = lax.axis_index("s") * nc + lax.axis_index("c")
        row0 = wid * rows_per_w
        pe0 = lax.rem(row0, seq_len)

        pltpu.sync_copy(ids_hbm.at[pl.ds(row0, rows_per_w)], idx_v)

        def chunk_body(c, _):
            base = c * _CHUNK
            g = pltpu.async_copy(
                table_hbm.at[idx_v.at[pl.ds(base, _CHUNK)]], rows_v, gsem
            )
            p = pltpu.async_copy(pe_hbm.at[pl.ds(pe0 + base, _CHUNK)], pe_v, psem)
            g.wait()
            p.wait()

            def mul_body(i, _):
                r = i // (d // 16)
                col = (i % (d // 16)) * 16
                rows_v[r, pl.ds(col, 16)] = (
                    rows_v[r, pl.ds(col, 16)] * pe_v[r, pl.ds(col, 16)]
                )
                return 0

            lax.fori_loop(0, _CHUNK * (d // 16), mul_body, 0)
            pltpu.sync_copy(rows_v, out_hbm.at[pl.ds(row0 + base, _CHUNK)])
            return 0

        lax.fori_loop(0, n_chunks, chunk_body, 0)

    return body


def kernel(input_ids, embedding_table):
    b, seq_len = input_ids.shape
    d = embedding_table.shape[1]
    n_rows = b * seq_len
    rows_per_w = n_rows // _NUM_WORKERS
    ids_flat = input_ids.reshape(n_rows).astype(jnp.int32)
    pe = jnp.asarray(_PE[:seq_len])
    out = _make_sc_kernel(n_rows, d, seq_len, rows_per_w)(
        ids_flat, embedding_table, pe
    )
    return out.reshape(b, seq_len, d)


# speedup vs baseline: 13.6053x; 2.9333x over previous
import functools

import jax
import jax.numpy as jnp
import numpy as np
from jax import lax
from jax.experimental import pallas as pl
from jax.experimental.pallas import tpu as pltpu
from jax.experimental.pallas import tpu_sc as plsc

MODEL_DIM = 2048
MAX_SEQ_LEN = 8192


def _pe_table_np(seq_len: int, model_dim: int) -> np.ndarray:
    positions = np.arange(0, seq_len, dtype=np.float32)[:, None]
    i = np.arange(0, model_dim // 2, dtype=np.float32)
    frequencies = np.power(np.float32(10000.0), 2.0 * i / np.float32(model_dim))
    pe = np.zeros((seq_len, model_dim), dtype=np.float32)
    pe[:, 0::2] = np.sin(positions / frequencies)
    pe[:, 1::2] = np.cos(positions / frequencies)
    return pe


_PE = _pe_table_np(MAX_SEQ_LEN, MODEL_DIM)

_NUM_WORKERS = 32
_P = 4
_NBUF = 2


def _make_sc_kernel(batch: int, seq_len: int, d: int):
    mesh = plsc.VectorSubcoreMesh(core_axis_name="c", subcore_axis_name="s")
    pos_per_w = seq_len // _NUM_WORKERS
    n_chunks = pos_per_w // _P
    n_rows = batch * seq_len
    dv = d // 16

    @functools.partial(
        pl.kernel,
        mesh=mesh,
        out_type=jax.ShapeDtypeStruct((n_rows, d), jnp.float32),
        scratch_types=[
            pltpu.VMEM((batch, pos_per_w), jnp.int32),
            pltpu.VMEM((_NBUF, batch, _P, d), jnp.float32),
            pltpu.VMEM((_NBUF, _P, d), jnp.float32),
            pltpu.SemaphoreType.DMA,
            pltpu.SemaphoreType.DMA,
            pltpu.SemaphoreType.DMA,
            pltpu.SemaphoreType.DMA,
        ],
    )
    def body(ids_hbm, table_hbm, pe_hbm, out_hbm, idx_v, rows_v, pe_v,
             gsem0, gsem1, ssem0, ssem1):
        nc = 2
        wid = lax.axis_index("s") * nc + lax.axis_index("c")
        pos0 = wid * pos_per_w

        for b in range(batch):
            pltpu.sync_copy(ids_hbm.at[pl.ds(b * seq_len + pos0, pos_per_w)],
                            idx_v.at[b])

        gsems = (gsem0, gsem1)
        ssems = (ssem0, ssem1)

        def load_descs(c, buf):
            descs = [
                pltpu.make_async_copy(
                    table_hbm.at[idx_v.at[b, pl.ds(c * _P, _P)]],
                    rows_v.at[buf, b], gsems[buf])
                for b in range(batch)
            ]
            descs.append(pltpu.make_async_copy(
                pe_hbm.at[pl.ds(pos0 + c * _P, _P)], pe_v.at[buf], gsems[buf]))
            return descs

        def store_descs(c, buf):
            return [
                pltpu.make_async_copy(
                    rows_v.at[buf, b],
                    out_hbm.at[pl.ds(b * seq_len + pos0 + c * _P, _P)],
                    ssems[buf])
                for b in range(batch)
            ]

        def multiply(buf):
            def mul_body(j, _):
                col = j * 16
                for p in range(_P):
                    pe_vec = pe_v[buf, p, pl.ds(col, 16)]
                    for b in range(batch):
                        rows_v[buf, b, p, pl.ds(col, 16)] = (
                            rows_v[buf, b, p, pl.ds(col, 16)] * pe_vec
                        )
                return 0

            lax.fori_loop(0, dv, mul_body, 0)

        for desc in load_descs(0, 0):
            desc.start()

        def pair_body(g, _):
            for local in range(_NBUF):
                c = _NBUF * g + local
                buf = local
                other = 1 - local

                @pl.when(c + 1 < n_chunks)
                def _():
                    @pl.when(c >= 1)
                    def _():
                        for desc in store_descs(c - 1, other):
                            desc.wait()

                    for desc in load_descs(c + 1, other):
                        desc.start()

                for desc in load_descs(c, buf):
                    desc.wait()
                multiply(buf)
                for desc in store_descs(c, buf):
                    desc.start()
            return 0

        lax.fori_loop(0, n_chunks // _NBUF, pair_body, 0)

        for desc in store_descs(n_chunks - 2, 0):
            desc.wait()
        for desc in store_descs(n_chunks - 1, 1):
            desc.wait()

    return body


def kernel(input_ids, embedding_table):
    b, seq_len = input_ids.shape
    d = embedding_table.shape[1]
    ids_flat = input_ids.reshape(b * seq_len).astype(jnp.int32)
    pe = jnp.asarray(_PE[:seq_len])
    out = _make_sc_kernel(b, seq_len, d)(ids_flat, embedding_table, pe)
    return out.reshape(b, seq_len, d)
